# baseline (speedup 1.0000x reference)
import functools

import jax
import jax.numpy as jnp
from jax import lax
from jax.experimental import pallas as pl
from jax.experimental.pallas import tpu as pltpu
from jax.experimental.pallas import tpu_sc as plsc

N = 10000
E = 320000
H = 128
A = 64
L = 3
R = 232
T = 182
TP = 184
NP = 10240
NW = 32
EW = E // NW
K = 40
NROWS = NP // 16


def _dotT(x, w):
    return lax.dot_general(x, w, (((1,), (1,)), ((), ())),
                           preferred_element_type=jnp.float32)


def _prep_body(rela_ref, tpad_ref, pw_ref, att1_ref,
               rp_ref, tp_ref, ra_ref, qa_ref):
    pw = pw_ref[...]
    tp_ref[...] = _dotT(tpad_ref[...], pw)
    for i in range(L):
        re = rela_ref[i]
        w1 = att1_ref[i]
        rp_ref[i] = _dotT(re, pw)
        ra_ref[i] = _dotT(re, w1[:, H:2 * H])
        qa_ref[i] = _dotT(re, w1[:, 2 * H:])


def _prep(rela, tpad, pw, att1):
    return pl.pallas_call(
        _prep_body,
        out_shape=(
            jax.ShapeDtypeStruct((L, R, H), jnp.float32),
            jax.ShapeDtypeStruct((TP, H), jnp.float32),
            jax.ShapeDtypeStruct((L, R, A), jnp.float32),
            jax.ShapeDtypeStruct((L, R, A), jnp.float32),
        ),
    )(rela, tpad, pw, att1)


def _tables_body(rp_ref, tp_ref, ra_ref, qa_ref, crt_ref, crq_ref):
    rp = rp_ref[0]
    tp = tp_ref[...]
    ra = ra_ref[0]
    qa = qa_ref[0]
    crt_ref[0] = rp[:, None, :] + tp[None, :, :]
    crq_ref[0] = ra[:, None, :] + qa[None, :, :]


def _tables(rp, tp, ra, qa):
    rb = 8
    grid = (L, R // rb)
    return pl.pallas_call(
        _tables_body,
        grid=grid,
        in_specs=[
            pl.BlockSpec((1, rb, H), lambda i, j: (i, j, 0)),
            pl.BlockSpec((TP, H), lambda i, j: (0, 0)),
            pl.BlockSpec((1, rb, A), lambda i, j: (i, j, 0)),
            pl.BlockSpec((1, R, A), lambda i, j: (i, 0, 0)),
        ],
        out_specs=(
            pl.BlockSpec((1, rb, TP, H), lambda i, j: (i, j, 0, 0)),
            pl.BlockSpec((1, rb, R, A), lambda i, j: (i, j, 0, 0)),
        ),
        out_shape=(
            jax.ShapeDtypeStruct((L, R, TP, H), jnp.float32),
            jax.ShapeDtypeStruct((L, R, R, A), jnp.float32),
        ),
    )(rp, tp, ra, qa)


def _hxf_body(p_ref, w3_ref, hx_ref):
    h = jnp.maximum(p_ref[0] + p_ref[1], 0.0)
    hx_ref[...] = _dotT(h, w3_ref[...])


def _hxf(p, w3):
    nb = 1000
    return pl.pallas_call(
        _hxf_body,
        grid=(N // nb,),
        in_specs=[
            pl.BlockSpec((2, nb, H), lambda j: (0, j, 0)),
            pl.BlockSpec((H + A, H), lambda j: (0, 0)),
        ],
        out_specs=pl.BlockSpec((nb, H + A), lambda j: (j, 0)),
        out_shape=jax.ShapeDtypeStruct((N, H + A), jnp.float32),
    )(p, w3)


def _final_body(p_ref, cw_ref, cb_ref, out_ref):
    h = jnp.maximum(p_ref[0] + p_ref[1], 0.0)
    out_ref[...] = lax.dot_general(h, cw_ref[...], (((1,), (0,)), ((), ())),
                                   preferred_element_type=jnp.float32) + cb_ref[0]


def _final(p, cw, cb):
    nb = 1000
    return pl.pallas_call(
        _final_body,
        grid=(N // nb,),
        in_specs=[
            pl.BlockSpec((2, nb, H), lambda j: (0, j, 0)),
            pl.BlockSpec((H, H), lambda j: (0, 0)),
            pl.BlockSpec((1,), lambda j: (0,)),
        ],
        out_specs=pl.BlockSpec((nb, H), lambda j: (j, 0)),
        out_shape=jax.ShapeDtypeStruct((N, H), jnp.float32),
    )(p, cw, cb)


NCH = EW // K


def _edge_body(pk_h, hx_h, crt_h, crq_h, att2_h, zer_h, p_h,
               pk0, pk1, a0, a1, hx0, hx1, m0, m1,
               att2_v, accum, gs0, gs1):
    cid = lax.axis_index("c")
    sid = lax.axis_index("s")
    wid = sid * 2 + cid

    pltpu.sync_copy(zer_h.at[pl.ds(sid * NROWS, NROWS)],
                    accum.at[pl.ds(sid * NROWS, NROWS)])
    pltpu.sync_copy(att2_h, att2_v)
    plsc.subcore_barrier()

    bufs = ((pk0, a0, hx0, m0, gs0), (pk1, a1, hx1, m1, gs1))

    def issue(c, b):
        pk, av, hxv, mv, gs = bufs[b]
        pltpu.sync_copy(pk_h.at[wid, c], pk)
        pltpu.async_copy(crq_h.at[pk.at[0]], av, gs)
        pltpu.async_copy(crt_h.at[pk.at[1]], mv, gs)
        pltpu.async_copy(hx_h.at[pk.at[2]], hxv, gs)

    def work(c, b):
        pk, av, hxv, mv, gs = bufs[b]
        pltpu.make_async_copy(crq_h.at[pk.at[0]], av, gs).wait()
        pltpu.make_async_copy(crt_h.at[pk.at[1]], mv, gs).wait()
        pltpu.make_async_copy(hx_h.at[pk.at[2]], hxv, gs).wait()

        @plsc.parallel_loop(0, K, unroll=8)
        def esc(e):
            t = jnp.zeros((16,), jnp.float32)
            for q in range(A // 16):
                sl = pl.ds(q * 16, 16)
                u = jnp.maximum(av[e, sl] + hxv[e, pl.ds(H + q * 16, 16)], 0.0)
                t = t + u * att2_v[sl]
            tot = jnp.sum(t)
            sv = jnp.full((16,), tot, jnp.float32)
            s = 1.0 / (1.0 + jnp.exp(-sv))
            for q in range(H // 16):
                sl = pl.ds(q * 16, 16)
                mv[e, sl] = (mv[e, sl] + hxv[e, sl]) * s
        pltpu.sync_copy(mv, accum.at[pk.at[3]], add=True)

    issue(0, 0)

    def pair(cc, carry):
        c0 = 2 * cc
        c1 = c0 + 1

        @pl.when(c1 < NCH)
        def _():
            issue(c1, 1)

        work(c0, 0)

        @pl.when(c0 + 2 < NCH)
        def _():
            issue(c0 + 2, 0)

        @pl.when(c1 < NCH)
        def _():
            work(c1, 1)

        return carry

    lax.fori_loop(0, (NCH + 1) // 2, pair, 0)
    plsc.subcore_barrier()
    pltpu.sync_copy(accum.at[pl.ds(sid * NROWS, NROWS)],
                    p_h.at[cid, pl.ds(sid * NROWS, NROWS)])


def _edge_pass(pk, hx, crt, crq, att2, zer):
    mesh = plsc.VectorSubcoreMesh(core_axis_name="c", subcore_axis_name="s")
    kfn = pl.kernel(
        _edge_body,
        out_type=jax.ShapeDtypeStruct((2, NP, H), jnp.float32),
        mesh=mesh,
        compiler_params=pltpu.CompilerParams(needs_layout_passes=False,
                                             use_tc_tiling_on_sc=False),
        scratch_types=[
            pltpu.VMEM((4, K), jnp.int32),
            pltpu.VMEM((4, K), jnp.int32),
            pltpu.VMEM((K, A), jnp.float32),
            pltpu.VMEM((K, A), jnp.float32),
            pltpu.VMEM((K, H + A), jnp.float32),
            pltpu.VMEM((K, H + A), jnp.float32),
            pltpu.VMEM((K, H), jnp.float32),
            pltpu.VMEM((K, H), jnp.float32),
            pltpu.VMEM((A,), jnp.float32),
            pltpu.VMEM_SHARED((NP, H), jnp.float32),
            pltpu.SemaphoreType.DMA,
            pltpu.SemaphoreType.DMA,
        ],
    )
    return kfn(pk, hx, crt, crq, att2, zer)


@jax.jit
def kernel(edge_src, edge_dst, edge_rel, rel_time, query_rel,
           rela_embed, time_embed, att1_w, att2_w, past_w, cls_w, cls_b):
    irt = edge_rel * TP + rel_time
    irq = edge_rel * R + query_rel
    tpad = jnp.pad(time_embed, ((0, TP - T), (0, 0)))

    rp, tp, ra, qa = _prep(rela_embed, tpad, past_w, att1_w)
    crt, crq = _tables(rp, tp, ra, qa)
    crt = crt.reshape(L, R * TP, H)
    crq = crq.reshape(L, R * R, A)

    zer = jnp.zeros((NP, H), jnp.float32)

    idx4 = jnp.stack([irq, irt, edge_src, edge_dst])
    pk = idx4.reshape(4, NW, NCH, K).transpose(1, 2, 0, 3)

    hx = jnp.zeros((N, H + A), jnp.float32)
    p = None
    for i in range(L):
        if i > 0:
            w3 = jnp.concatenate([past_w, att1_w[i][:, :H]], axis=0)
            hx = _hxf(p, w3)
        p = _edge_pass(pk, hx, crt[i], crq[i], att2_w[i].reshape(A), zer)

    cls_rep = jnp.broadcast_to(cls_w.reshape(H, 1), (H, H))
    return _final(p, cls_rep, cls_b)[:, 0]


# speedup vs baseline: 1.0573x; 1.0573x over previous
import functools

import jax
import jax.numpy as jnp
from jax import lax
from jax.experimental import pallas as pl
from jax.experimental.pallas import tpu as pltpu
from jax.experimental.pallas import tpu_sc as plsc

N = 10000
E = 320000
H = 128
A = 64
L = 3
R = 232
T = 182
TP = 184
NP = 10240
NW = 32
EW = E // NW
K = 40
NROWS = NP // 16


def _dotT(x, w):
    return lax.dot_general(x, w, (((1,), (1,)), ((), ())),
                           precision=lax.Precision.HIGHEST,
                           preferred_element_type=jnp.float32)


def _prep_body(rela_ref, tpad_ref, pw_ref, att1_ref,
               rp_ref, tp_ref, ra_ref, qa_ref):
    pw = pw_ref[...]
    tp_ref[...] = _dotT(tpad_ref[...], pw)
    for i in range(L):
        re = rela_ref[i]
        w1 = att1_ref[i]
        rp_ref[i] = _dotT(re, pw)
        ra_ref[i] = _dotT(re, w1[:, H:2 * H])
        qa_ref[i] = _dotT(re, w1[:, 2 * H:])


def _prep(rela, tpad, pw, att1):
    return pl.pallas_call(
        _prep_body,
        out_shape=(
            jax.ShapeDtypeStruct((L, R, H), jnp.float32),
            jax.ShapeDtypeStruct((TP, H), jnp.float32),
            jax.ShapeDtypeStruct((L, R, A), jnp.float32),
            jax.ShapeDtypeStruct((L, R, A), jnp.float32),
        ),
    )(rela, tpad, pw, att1)


def _tables_body(rp_ref, tp_ref, ra_ref, qa_ref, crt_ref, crq_ref):
    rp = rp_ref[0]
    tp = tp_ref[...]
    ra = ra_ref[0]
    qa = qa_ref[0]
    crt_ref[0] = rp[:, None, :] + tp[None, :, :]
    crq_ref[0] = ra[:, None, :] + qa[None, :, :]


def _tables(rp, tp, ra, qa):
    rb = 8
    grid = (L, R // rb)
    return pl.pallas_call(
        _tables_body,
        grid=grid,
        in_specs=[
            pl.BlockSpec((1, rb, H), lambda i, j: (i, j, 0)),
            pl.BlockSpec((TP, H), lambda i, j: (0, 0)),
            pl.BlockSpec((1, rb, A), lambda i, j: (i, j, 0)),
            pl.BlockSpec((1, R, A), lambda i, j: (i, 0, 0)),
        ],
        out_specs=(
            pl.BlockSpec((1, rb, TP, H), lambda i, j: (i, j, 0, 0)),
            pl.BlockSpec((1, rb, R, A), lambda i, j: (i, j, 0, 0)),
        ),
        out_shape=(
            jax.ShapeDtypeStruct((L, R, TP, H), jnp.float32),
            jax.ShapeDtypeStruct((L, R, R, A), jnp.float32),
        ),
    )(rp, tp, ra, qa)


def _hxf_body(p_ref, w3_ref, hx_ref):
    h = jnp.maximum(p_ref[0] + p_ref[1], 0.0)
    hx_ref[...] = _dotT(h, w3_ref[...])


def _hxf(p, w3):
    nb = 1000
    return pl.pallas_call(
        _hxf_body,
        grid=(N // nb,),
        in_specs=[
            pl.BlockSpec((2, nb, H), lambda j: (0, j, 0)),
            pl.BlockSpec((H + A, H), lambda j: (0, 0)),
        ],
        out_specs=pl.BlockSpec((nb, H + A), lambda j: (j, 0)),
        out_shape=jax.ShapeDtypeStruct((N, H + A), jnp.float32),
    )(p, w3)


def _final_body(p_ref, cw_ref, cb_ref, out_ref):
    h = jnp.maximum(p_ref[0] + p_ref[1], 0.0)
    out_ref[...] = lax.dot_general(h, cw_ref[...], (((1,), (0,)), ((), ())),
                                   precision=lax.Precision.HIGHEST,
                                   preferred_element_type=jnp.float32) + cb_ref[0]


def _final(p, cw, cb):
    nb = 1000
    return pl.pallas_call(
        _final_body,
        grid=(N // nb,),
        in_specs=[
            pl.BlockSpec((2, nb, H), lambda j: (0, j, 0)),
            pl.BlockSpec((H, H), lambda j: (0, 0)),
            pl.BlockSpec((1,), lambda j: (0,)),
        ],
        out_specs=pl.BlockSpec((nb, H), lambda j: (j, 0)),
        out_shape=jax.ShapeDtypeStruct((N, H), jnp.float32),
    )(p, cw, cb)


def _s0_body(crq_ref, a2_ref, out_ref):
    u = jnp.maximum(crq_ref[...], 0.0)
    t = lax.dot_general(u, a2_ref[...], (((1,), (0,)), ((), ())),
                        precision=lax.Precision.HIGHEST,
                        preferred_element_type=jnp.float32)
    out_ref[...] = 1.0 / (1.0 + jnp.exp(-t))


def _s0(crq0, a2rep):
    nb = 6728
    return pl.pallas_call(
        _s0_body,
        grid=(R * R // nb,),
        in_specs=[
            pl.BlockSpec((nb, A), lambda j: (j, 0)),
            pl.BlockSpec((A, 16), lambda j: (0, 0)),
        ],
        out_specs=pl.BlockSpec((nb, 16), lambda j: (j, 0)),
        out_shape=jax.ShapeDtypeStruct((R * R, 16), jnp.float32),
    )(crq0, a2rep)


NCH = EW // K


def _edge_body(pk_h, hx_h, crt_h, crq_h, att2_h, zer_h, p_h,
               pk0, pk1, a0, a1, hx0, hx1, m0, m1,
               att2_v, accum, gs0, gs1):
    cid = lax.axis_index("c")
    sid = lax.axis_index("s")
    wid = sid * 2 + cid

    pltpu.sync_copy(zer_h.at[pl.ds(sid * NROWS, NROWS)],
                    accum.at[pl.ds(sid * NROWS, NROWS)])
    pltpu.sync_copy(att2_h, att2_v)
    plsc.subcore_barrier()

    bufs = ((pk0, a0, hx0, m0, gs0), (pk1, a1, hx1, m1, gs1))

    def issue(c, b):
        pk, av, hxv, mv, gs = bufs[b]
        pltpu.sync_copy(pk_h.at[wid, c], pk)
        pltpu.async_copy(crq_h.at[pk.at[0]], av, gs)
        pltpu.async_copy(crt_h.at[pk.at[1]], mv, gs)
        pltpu.async_copy(hx_h.at[pk.at[2]], hxv, gs)

    def work(c, b):
        pk, av, hxv, mv, gs = bufs[b]
        pltpu.make_async_copy(crq_h.at[pk.at[0]], av, gs).wait()
        pltpu.make_async_copy(crt_h.at[pk.at[1]], mv, gs).wait()
        pltpu.make_async_copy(hx_h.at[pk.at[2]], hxv, gs).wait()

        @plsc.parallel_loop(0, K, unroll=4)
        def esc(e):
            t = jnp.zeros((16,), jnp.float32)
            for q in range(A // 16):
                sl = pl.ds(q * 16, 16)
                u = jnp.maximum(av[e, sl] + hxv[e, pl.ds(H + q * 16, 16)], 0.0)
                t = t + u * att2_v[sl]
            tot = jnp.sum(t)
            sv = jnp.full((16,), tot, jnp.float32)
            s = 1.0 / (1.0 + jnp.exp(-sv))
            for q in range(H // 16):
                sl = pl.ds(q * 16, 16)
                mv[e, sl] = (mv[e, sl] + hxv[e, sl]) * s
        pltpu.sync_copy(mv, accum.at[pk.at[3]], add=True)

    issue(0, 0)

    def pair(cc, carry):
        c0 = 2 * cc
        c1 = c0 + 1

        @pl.when(c1 < NCH)
        def _():
            issue(c1, 1)

        work(c0, 0)

        @pl.when(c0 + 2 < NCH)
        def _():
            issue(c0 + 2, 0)

        @pl.when(c1 < NCH)
        def _():
            work(c1, 1)

        return carry

    lax.fori_loop(0, (NCH + 1) // 2, pair, 0)
    plsc.subcore_barrier()
    pltpu.sync_copy(accum.at[pl.ds(sid * NROWS, NROWS)],
                    p_h.at[cid, pl.ds(sid * NROWS, NROWS)])


def _edge_pass(pk, hx, crt, crq, att2, zer):
    mesh = plsc.VectorSubcoreMesh(core_axis_name="c", subcore_axis_name="s")
    kfn = pl.kernel(
        _edge_body,
        out_type=jax.ShapeDtypeStruct((2, NP, H), jnp.float32),
        mesh=mesh,
        compiler_params=pltpu.CompilerParams(needs_layout_passes=False,
                                             use_tc_tiling_on_sc=False),
        scratch_types=[
            pltpu.VMEM((4, K), jnp.int32),
            pltpu.VMEM((4, K), jnp.int32),
            pltpu.VMEM((K, A), jnp.float32),
            pltpu.VMEM((K, A), jnp.float32),
            pltpu.VMEM((K, H + A), jnp.float32),
            pltpu.VMEM((K, H + A), jnp.float32),
            pltpu.VMEM((K, H), jnp.float32),
            pltpu.VMEM((K, H), jnp.float32),
            pltpu.VMEM((A,), jnp.float32),
            pltpu.VMEM_SHARED((NP, H), jnp.float32),
            pltpu.SemaphoreType.DMA,
            pltpu.SemaphoreType.DMA,
        ],
    )
    return kfn(pk, hx, crt, crq, att2, zer)


def _edge0_body(pk_h, s0_h, crt_h, zer_h, p_h,
                pk0, pk1, s0b0, s0b1, m0, m1, accum, gs0, gs1):
    cid = lax.axis_index("c")
    sid = lax.axis_index("s")
    wid = sid * 2 + cid

    pltpu.sync_copy(zer_h.at[pl.ds(sid * NROWS, NROWS)],
                    accum.at[pl.ds(sid * NROWS, NROWS)])
    plsc.subcore_barrier()

    bufs = ((pk0, s0b0, m0, gs0), (pk1, s0b1, m1, gs1))

    def issue(c, b):
        pk, sv, mv, gs = bufs[b]
        pltpu.sync_copy(pk_h.at[wid, c], pk)
        pltpu.async_copy(s0_h.at[pk.at[0]], sv, gs)
        pltpu.async_copy(crt_h.at[pk.at[1]], mv, gs)

    def work(c, b):
        pk, sv, mv, gs = bufs[b]
        pltpu.make_async_copy(s0_h.at[pk.at[0]], sv, gs).wait()
        pltpu.make_async_copy(crt_h.at[pk.at[1]], mv, gs).wait()

        @plsc.parallel_loop(0, K, unroll=4)
        def esc(e):
            s = sv[e, pl.ds(0, 16)]
            for q in range(H // 16):
                sl = pl.ds(q * 16, 16)
                mv[e, sl] = mv[e, sl] * s

        pltpu.sync_copy(mv, accum.at[pk.at[3]], add=True)

    issue(0, 0)

    def pair(cc, carry):
        c0 = 2 * cc
        c1 = c0 + 1

        @pl.when(c1 < NCH)
        def _():
            issue(c1, 1)

        work(c0, 0)

        @pl.when(c0 + 2 < NCH)
        def _():
            issue(c0 + 2, 0)

        @pl.when(c1 < NCH)
        def _():
            work(c1, 1)

        return carry

    lax.fori_loop(0, (NCH + 1) // 2, pair, 0)
    plsc.subcore_barrier()
    pltpu.sync_copy(accum.at[pl.ds(sid * NROWS, NROWS)],
                    p_h.at[cid, pl.ds(sid * NROWS, NROWS)])


def _edge0_pass(pk, s0rep, crt0, zer):
    mesh = plsc.VectorSubcoreMesh(core_axis_name="c", subcore_axis_name="s")
    kfn = pl.kernel(
        _edge0_body,
        out_type=jax.ShapeDtypeStruct((2, NP, H), jnp.float32),
        mesh=mesh,
        compiler_params=pltpu.CompilerParams(needs_layout_passes=False,
                                             use_tc_tiling_on_sc=False),
        scratch_types=[
            pltpu.VMEM((4, K), jnp.int32),
            pltpu.VMEM((4, K), jnp.int32),
            pltpu.VMEM((K, 16), jnp.float32),
            pltpu.VMEM((K, 16), jnp.float32),
            pltpu.VMEM((K, H), jnp.float32),
            pltpu.VMEM((K, H), jnp.float32),
            pltpu.VMEM_SHARED((NP, H), jnp.float32),
            pltpu.SemaphoreType.DMA,
            pltpu.SemaphoreType.DMA,
        ],
    )
    return kfn(pk, s0rep, crt0, zer)


@jax.jit
def kernel(edge_src, edge_dst, edge_rel, rel_time, query_rel,
           rela_embed, time_embed, att1_w, att2_w, past_w, cls_w, cls_b):
    irt = edge_rel * TP + rel_time
    irq = edge_rel * R + query_rel
    tpad = jnp.pad(time_embed, ((0, TP - T), (0, 0)))

    rp, tp, ra, qa = _prep(rela_embed, tpad, past_w, att1_w)
    crt, crq = _tables(rp, tp, ra, qa)
    crt = crt.reshape(L, R * TP, H)
    crq = crq.reshape(L, R * R, A)

    zer = jnp.zeros((NP, H), jnp.float32)

    idx4 = jnp.stack([irq, irt, edge_src, edge_dst])
    pk = idx4.reshape(4, NW, NCH, K).transpose(1, 2, 0, 3)

    a2rep = jnp.broadcast_to(att2_w[0].reshape(A, 1), (A, 16))
    s0rep = _s0(crq[0], a2rep)
    p = _edge0_pass(pk, s0rep, crt[0], zer)
    for i in range(1, L):
        w3 = jnp.concatenate([past_w, att1_w[i][:, :H]], axis=0)
        hx = _hxf(p, w3)
        p = _edge_pass(pk, hx, crt[i], crq[i], att2_w[i].reshape(A), zer)

    cls_rep = jnp.broadcast_to(cls_w.reshape(H, 1), (H, H))
    return _final(p, cls_rep, cls_b)[:, 0]


# speedup vs baseline: 1.0920x; 1.0329x over previous
import functools

import jax
import jax.numpy as jnp
from jax import lax
from jax.experimental import pallas as pl
from jax.experimental.pallas import tpu as pltpu
from jax.experimental.pallas import tpu_sc as plsc

N = 10000
E = 320000
H = 128
A = 64
L = 3
R = 232
T = 182
TP = 184
NP = 10112
NW = 32
EW = E // NW
K = 64
NCH = -(-EW // K)
EWP = NCH * K
NROWS = NP // 16


def _dotT(x, w):
    return lax.dot_general(x, w, (((1,), (1,)), ((), ())),
                           precision=lax.Precision.HIGHEST,
                           preferred_element_type=jnp.float32)


def _prep_body(rela_ref, tpad_ref, pw_ref, att1_ref,
               rp_ref, tp_ref, ra_ref, qa_ref):
    pw = pw_ref[...]
    tp_ref[...] = _dotT(tpad_ref[...], pw)
    for i in range(L):
        re = rela_ref[i]
        w1 = att1_ref[i]
        rp_ref[i] = _dotT(re, pw)
        ra_ref[i] = _dotT(re, w1[:, H:2 * H])
        qa_ref[i] = _dotT(re, w1[:, 2 * H:])


def _prep(rela, tpad, pw, att1):
    return pl.pallas_call(
        _prep_body,
        out_shape=(
            jax.ShapeDtypeStruct((L, R, H), jnp.float32),
            jax.ShapeDtypeStruct((TP, H), jnp.float32),
            jax.ShapeDtypeStruct((L, R, A), jnp.float32),
            jax.ShapeDtypeStruct((L, R, A), jnp.float32),
        ),
    )(rela, tpad, pw, att1)


def _tables_body(rp_ref, tp_ref, ra_ref, qa_ref, crt_ref, crq_ref):
    rp = rp_ref[0]
    tp = tp_ref[...]
    ra = ra_ref[0]
    qa = qa_ref[0]
    crt_ref[0] = rp[:, None, :] + tp[None, :, :]
    crq_ref[0] = ra[:, None, :] + qa[None, :, :]


def _tables(rp, tp, ra, qa):
    rb = 8
    grid = (L, R // rb)
    return pl.pallas_call(
        _tables_body,
        grid=grid,
        in_specs=[
            pl.BlockSpec((1, rb, H), lambda i, j: (i, j, 0)),
            pl.BlockSpec((TP, H), lambda i, j: (0, 0)),
            pl.BlockSpec((1, rb, A), lambda i, j: (i, j, 0)),
            pl.BlockSpec((1, R, A), lambda i, j: (i, 0, 0)),
        ],
        out_specs=(
            pl.BlockSpec((1, rb, TP, H), lambda i, j: (i, j, 0, 0)),
            pl.BlockSpec((1, rb, R, A), lambda i, j: (i, j, 0, 0)),
        ),
        out_shape=(
            jax.ShapeDtypeStruct((L, R, TP, H), jnp.float32),
            jax.ShapeDtypeStruct((L, R, R, A), jnp.float32),
        ),
    )(rp, tp, ra, qa)


def _hxf_body(p_ref, w3_ref, hx_ref):
    h = jnp.maximum(p_ref[0] + p_ref[1], 0.0)
    hx_ref[...] = _dotT(h, w3_ref[...])


def _hxf(p, w3):
    nb = 1000
    return pl.pallas_call(
        _hxf_body,
        grid=(N // nb,),
        in_specs=[
            pl.BlockSpec((2, nb, H), lambda j: (0, j, 0)),
            pl.BlockSpec((H + A, H), lambda j: (0, 0)),
        ],
        out_specs=pl.BlockSpec((nb, H + A), lambda j: (j, 0)),
        out_shape=jax.ShapeDtypeStruct((N, H + A), jnp.float32),
    )(p, w3)


def _final_body(p_ref, cw_ref, cb_ref, out_ref):
    h = jnp.maximum(p_ref[0] + p_ref[1], 0.0)
    out_ref[...] = lax.dot_general(h, cw_ref[...], (((1,), (0,)), ((), ())),
                                   precision=lax.Precision.HIGHEST,
                                   preferred_element_type=jnp.float32) + cb_ref[0]


def _final(p, cw, cb):
    nb = 1000
    return pl.pallas_call(
        _final_body,
        grid=(N // nb,),
        in_specs=[
            pl.BlockSpec((2, nb, H), lambda j: (0, j, 0)),
            pl.BlockSpec((H, H), lambda j: (0, 0)),
            pl.BlockSpec((1,), lambda j: (0,)),
        ],
        out_specs=pl.BlockSpec((nb, H), lambda j: (j, 0)),
        out_shape=jax.ShapeDtypeStruct((N, H), jnp.float32),
    )(p, cw, cb)


def _s0_body(crq_ref, a2_ref, out_ref):
    u = jnp.maximum(crq_ref[...], 0.0)
    t = lax.dot_general(u, a2_ref[...], (((1,), (0,)), ((), ())),
                        precision=lax.Precision.HIGHEST,
                        preferred_element_type=jnp.float32)
    out_ref[...] = 1.0 / (1.0 + jnp.exp(-t))


def _s0(crq0, a2rep):
    nb = 6728
    return pl.pallas_call(
        _s0_body,
        grid=(R * R // nb,),
        in_specs=[
            pl.BlockSpec((nb, A), lambda j: (j, 0)),
            pl.BlockSpec((A, 16), lambda j: (0, 0)),
        ],
        out_specs=pl.BlockSpec((nb, 16), lambda j: (j, 0)),
        out_shape=jax.ShapeDtypeStruct((R * R, 16), jnp.float32),
    )(crq0, a2rep)


def _edge_body(pk_h, hx_h, crt_h, crq_h, att2_h, zer_h, p_h,
               pk0, pk1, a0, a1, hx0, hx1, m0, m1,
               att2_v, accum, gs0, gs1):
    cid = lax.axis_index("c")
    sid = lax.axis_index("s")
    wid = sid * 2 + cid

    pltpu.sync_copy(zer_h.at[pl.ds(sid * NROWS, NROWS)],
                    accum.at[pl.ds(sid * NROWS, NROWS)])
    pltpu.sync_copy(att2_h, att2_v)
    plsc.subcore_barrier()

    bufs = ((pk0, a0, hx0, m0, gs0), (pk1, a1, hx1, m1, gs1))

    def issue(c, b):
        pk, av, hxv, mv, gs = bufs[b]
        pltpu.sync_copy(pk_h.at[wid, c], pk)
        pltpu.async_copy(crq_h.at[pk.at[0]], av, gs)
        pltpu.async_copy(crt_h.at[pk.at[1]], mv, gs)
        pltpu.async_copy(hx_h.at[pk.at[2]], hxv, gs)

    def work(c, b):
        pk, av, hxv, mv, gs = bufs[b]
        pltpu.make_async_copy(crq_h.at[pk.at[0]], av, gs).wait()
        pltpu.make_async_copy(crt_h.at[pk.at[1]], mv, gs).wait()
        pltpu.make_async_copy(hx_h.at[pk.at[2]], hxv, gs).wait()

        @plsc.parallel_loop(0, K, unroll=4)
        def esc(e):
            t = jnp.zeros((16,), jnp.float32)
            for q in range(A // 16):
                sl = pl.ds(q * 16, 16)
                u = jnp.maximum(av[e, sl] + hxv[e, pl.ds(H + q * 16, 16)], 0.0)
                t = t + u * att2_v[sl]
            tot = jnp.sum(t)
            sv = jnp.full((16,), tot, jnp.float32)
            s = 1.0 / (1.0 + jnp.exp(-sv))
            for q in range(H // 16):
                sl = pl.ds(q * 16, 16)
                mv[e, sl] = (mv[e, sl] + hxv[e, sl]) * s
        pltpu.sync_copy(mv, accum.at[pk.at[3]], add=True)

    issue(0, 0)

    def pair(cc, carry):
        c0 = 2 * cc
        c1 = c0 + 1

        @pl.when(c1 < NCH)
        def _():
            issue(c1, 1)

        work(c0, 0)

        @pl.when(c0 + 2 < NCH)
        def _():
            issue(c0 + 2, 0)

        @pl.when(c1 < NCH)
        def _():
            work(c1, 1)

        return carry

    lax.fori_loop(0, (NCH + 1) // 2, pair, 0)
    plsc.subcore_barrier()
    pltpu.sync_copy(accum.at[pl.ds(sid * NROWS, NROWS)],
                    p_h.at[cid, pl.ds(sid * NROWS, NROWS)])


def _edge_pass(pk, hx, crt, crq, att2, zer):
    mesh = plsc.VectorSubcoreMesh(core_axis_name="c", subcore_axis_name="s")
    kfn = pl.kernel(
        _edge_body,
        out_type=jax.ShapeDtypeStruct((2, NP, H), jnp.float32),
        mesh=mesh,
        compiler_params=pltpu.CompilerParams(needs_layout_passes=False,
                                             use_tc_tiling_on_sc=False),
        scratch_types=[
            pltpu.VMEM((4, K), jnp.int32),
            pltpu.VMEM((4, K), jnp.int32),
            pltpu.VMEM((K, A), jnp.float32),
            pltpu.VMEM((K, A), jnp.float32),
            pltpu.VMEM((K, H + A), jnp.float32),
            pltpu.VMEM((K, H + A), jnp.float32),
            pltpu.VMEM((K, H), jnp.float32),
            pltpu.VMEM((K, H), jnp.float32),
            pltpu.VMEM((A,), jnp.float32),
            pltpu.VMEM_SHARED((NP, H), jnp.float32),
            pltpu.SemaphoreType.DMA,
            pltpu.SemaphoreType.DMA,
        ],
    )
    return kfn(pk, hx, crt, crq, att2, zer)


def _edge0_body(pk_h, s0_h, crt_h, zer_h, p_h,
                pk0, pk1, s0b0, s0b1, m0, m1, accum, gs0, gs1):
    cid = lax.axis_index("c")
    sid = lax.axis_index("s")
    wid = sid * 2 + cid

    pltpu.sync_copy(zer_h.at[pl.ds(sid * NROWS, NROWS)],
                    accum.at[pl.ds(sid * NROWS, NROWS)])
    plsc.subcore_barrier()

    bufs = ((pk0, s0b0, m0, gs0), (pk1, s0b1, m1, gs1))

    def issue(c, b):
        pk, sv, mv, gs = bufs[b]
        pltpu.sync_copy(pk_h.at[wid, c], pk)
        pltpu.async_copy(s0_h.at[pk.at[0]], sv, gs)
        pltpu.async_copy(crt_h.at[pk.at[1]], mv, gs)

    def work(c, b):
        pk, sv, mv, gs = bufs[b]
        pltpu.make_async_copy(s0_h.at[pk.at[0]], sv, gs).wait()
        pltpu.make_async_copy(crt_h.at[pk.at[1]], mv, gs).wait()

        @plsc.parallel_loop(0, K, unroll=4)
        def esc(e):
            s = sv[e, pl.ds(0, 16)]
            for q in range(H // 16):
                sl = pl.ds(q * 16, 16)
                mv[e, sl] = mv[e, sl] * s

        pltpu.sync_copy(mv, accum.at[pk.at[3]], add=True)

    issue(0, 0)

    def pair(cc, carry):
        c0 = 2 * cc
        c1 = c0 + 1

        @pl.when(c1 < NCH)
        def _():
            issue(c1, 1)

        work(c0, 0)

        @pl.when(c0 + 2 < NCH)
        def _():
            issue(c0 + 2, 0)

        @pl.when(c1 < NCH)
        def _():
            work(c1, 1)

        return carry

    lax.fori_loop(0, (NCH + 1) // 2, pair, 0)
    plsc.subcore_barrier()
    pltpu.sync_copy(accum.at[pl.ds(sid * NROWS, NROWS)],
                    p_h.at[cid, pl.ds(sid * NROWS, NROWS)])


def _edge0_pass(pk, s0rep, crt0, zer):
    mesh = plsc.VectorSubcoreMesh(core_axis_name="c", subcore_axis_name="s")
    kfn = pl.kernel(
        _edge0_body,
        out_type=jax.ShapeDtypeStruct((2, NP, H), jnp.float32),
        mesh=mesh,
        compiler_params=pltpu.CompilerParams(needs_layout_passes=False,
                                             use_tc_tiling_on_sc=False),
        scratch_types=[
            pltpu.VMEM((4, K), jnp.int32),
            pltpu.VMEM((4, K), jnp.int32),
            pltpu.VMEM((K, 16), jnp.float32),
            pltpu.VMEM((K, 16), jnp.float32),
            pltpu.VMEM((K, H), jnp.float32),
            pltpu.VMEM((K, H), jnp.float32),
            pltpu.VMEM_SHARED((NP, H), jnp.float32),
            pltpu.SemaphoreType.DMA,
            pltpu.SemaphoreType.DMA,
        ],
    )
    return kfn(pk, s0rep, crt0, zer)


@jax.jit
def kernel(edge_src, edge_dst, edge_rel, rel_time, query_rel,
           rela_embed, time_embed, att1_w, att2_w, past_w, cls_w, cls_b):
    irt = edge_rel * TP + rel_time
    irq = edge_rel * R + query_rel
    tpad = jnp.pad(time_embed, ((0, TP - T), (0, 0)))

    rp, tp, ra, qa = _prep(rela_embed, tpad, past_w, att1_w)
    crt, crq = _tables(rp, tp, ra, qa)
    crt = crt.reshape(L, R * TP, H)
    crq = crq.reshape(L, R * R, A)

    zer = jnp.zeros((NP, H), jnp.float32)

    idx4 = jnp.stack([irq, irt, edge_src, edge_dst]).reshape(4, NW, EW)
    idx4 = jnp.pad(idx4, ((0, 0), (0, 0), (0, EWP - EW)))
    idx4 = idx4.at[3, :, EW:].set(NP - 1)
    pk = idx4.reshape(4, NW, NCH, K).transpose(1, 2, 0, 3)

    a2rep = jnp.broadcast_to(att2_w[0].reshape(A, 1), (A, 16))
    s0rep = _s0(crq[0], a2rep)
    p = _edge0_pass(pk, s0rep, crt[0], zer)
    for i in range(1, L):
        w3 = jnp.concatenate([past_w, att1_w[i][:, :H]], axis=0)
        hx = _hxf(p, w3)
        p = _edge_pass(pk, hx, crt[i], crq[i], att2_w[i].reshape(A), zer)

    cls_rep = jnp.broadcast_to(cls_w.reshape(H, 1), (H, H))
    return _final(p, cls_rep, cls_b)[:, 0]
